# baseline (speedup 1.0000x reference)
import functools

import jax
import jax.numpy as jnp
from jax.experimental import pallas as pl
from jax.experimental.pallas import tpu as pltpu

_TN = 1024


def _prep_body(wg_ref, bg_ref, w1_ref, b1_ref, w2_ref, b2_ref,
               wgb_ref, bgr_ref, w1t_ref, b1r_ref, w2r_ref, b2b_ref,
               sel_ref, *, n_exp, f_hid):
    ef = n_exp * f_hid
    wgb_ref[...] = wg_ref[...].astype(jnp.bfloat16)
    bgr_ref[...] = bg_ref[...].reshape(1, n_exp)
    w1t_ref[...] = jnp.concatenate(
        [w1_ref[e] for e in range(n_exp)], axis=1).astype(jnp.bfloat16)
    b1r_ref[...] = jnp.concatenate(
        [b1_ref[e] for e in range(n_exp)],
        axis=0).reshape(1, ef).astype(jnp.bfloat16)
    w2r_ref[...] = w2_ref[...].reshape(ef, w2_ref.shape[-1]).astype(
        jnp.bfloat16)
    b2b_ref[...] = b2_ref[...].astype(jnp.bfloat16)
    rr = jax.lax.broadcasted_iota(jnp.int32, (n_exp, ef), 0)
    cc = jax.lax.broadcasted_iota(jnp.int32, (n_exp, ef), 1)
    sel_ref[...] = (cc // f_hid == rr).astype(jnp.bfloat16)


def _moe_body(x_ref, wgb_ref, bgr_ref, w1t_ref, b1r_ref, w2r_ref, b2b_ref,
              sel_ref, o_ref):
    xb = x_ref[...].astype(jnp.bfloat16)
    logits = jnp.dot(xb, wgb_ref[...], preferred_element_type=jnp.float32)
    logits = logits + bgr_ref[...]
    m = jnp.max(logits, axis=1, keepdims=True)
    p = jnp.exp(logits - m)
    g = p / jnp.sum(p, axis=1, keepdims=True)
    gb = g.astype(jnp.bfloat16)

    h32 = jnp.dot(xb, w1t_ref[...], preferred_element_type=jnp.float32)
    h = jnp.maximum(h32.astype(jnp.bfloat16) + b1r_ref[...],
                    jnp.bfloat16(0.0))

    ge = jnp.dot(gb, sel_ref[...],
                 preferred_element_type=jnp.float32).astype(jnp.bfloat16)

    out = jnp.dot(h * ge, w2r_ref[...], preferred_element_type=jnp.float32)
    out = out + jnp.dot(gb, b2b_ref[...], preferred_element_type=jnp.float32)
    o_ref[...] = out


def kernel(x, Wg, bg, W1, b1, W2, b2):
    n, d = x.shape
    e, _, f = W1.shape
    ef = e * f
    bf = jnp.bfloat16
    wgb, bgr, w1t, b1r, w2r, b2b, sel = pl.pallas_call(
        functools.partial(_prep_body, n_exp=e, f_hid=f),
        out_shape=[
            jax.ShapeDtypeStruct((d, e), bf),
            jax.ShapeDtypeStruct((1, e), jnp.float32),
            jax.ShapeDtypeStruct((d, ef), bf),
            jax.ShapeDtypeStruct((1, ef), bf),
            jax.ShapeDtypeStruct((ef, d), bf),
            jax.ShapeDtypeStruct((e, d), bf),
            jax.ShapeDtypeStruct((e, ef), bf),
        ],
    )(Wg, bg, W1, b1, W2, b2)

    const = lambda i: (0, 0)
    return pl.pallas_call(
        _moe_body,
        grid=(n // _TN,),
        in_specs=[
            pl.BlockSpec((_TN, d), lambda i: (i, 0)),
            pl.BlockSpec((d, e), const),
            pl.BlockSpec((1, e), const),
            pl.BlockSpec((d, ef), const),
            pl.BlockSpec((1, ef), const),
            pl.BlockSpec((ef, d), const),
            pl.BlockSpec((e, d), const),
            pl.BlockSpec((e, ef), const),
        ],
        out_specs=pl.BlockSpec((_TN, d), lambda i: (i, 0)),
        out_shape=jax.ShapeDtypeStruct((n, d), x.dtype),
        compiler_params=pltpu.CompilerParams(
            dimension_semantics=("parallel",)),
    )(x, wgb, bgr, w1t, b1r, w2r, b2b, sel)


# speedup vs baseline: 1.0818x; 1.0818x over previous
import functools

import jax
import jax.numpy as jnp
from jax.experimental import pallas as pl
from jax.experimental.pallas import tpu as pltpu

_TN = 2048


def _moe_body(x_ref, wg_ref, bg_ref, w1_ref, b1_ref, w2_ref, b2_ref, o_ref,
              *, n_exp, f_hid):
    ef = n_exp * f_hid
    xb = x_ref[...].astype(jnp.bfloat16)

    logits = jnp.dot(xb, wg_ref[...].astype(jnp.bfloat16),
                     preferred_element_type=jnp.float32)
    logits = logits + bg_ref[...]
    m = jnp.max(logits, axis=1, keepdims=True)
    p = jnp.exp(logits - m)
    g = p / jnp.sum(p, axis=1, keepdims=True)
    gb = g.astype(jnp.bfloat16)

    w1t = jnp.concatenate(
        [w1_ref[e] for e in range(n_exp)], axis=1).astype(jnp.bfloat16)
    h32 = jnp.dot(xb, w1t, preferred_element_type=jnp.float32)
    h = jnp.maximum(h32.astype(jnp.bfloat16) + b1_ref[...].astype(jnp.bfloat16),
                    jnp.bfloat16(0.0))

    rr = jax.lax.broadcasted_iota(jnp.int32, (n_exp, ef), 0)
    cc = jax.lax.broadcasted_iota(jnp.int32, (n_exp, ef), 1)
    sel = (cc // f_hid == rr).astype(jnp.bfloat16)
    ge = jnp.dot(gb, sel,
                 preferred_element_type=jnp.float32).astype(jnp.bfloat16)

    out = jnp.dot(h * ge, w2_ref[...].astype(jnp.bfloat16),
                  preferred_element_type=jnp.float32)
    out = out + jnp.dot(gb, b2_ref[...].astype(jnp.bfloat16),
                        preferred_element_type=jnp.float32)
    o_ref[...] = out


def kernel(x, Wg, bg, W1, b1, W2, b2):
    n, d = x.shape
    e, _, f = W1.shape
    ef = e * f
    bgr = bg.reshape(1, e)
    b1r = b1.reshape(1, ef)
    w2r = W2.reshape(ef, d)
    const = lambda i: (0, 0)
    return pl.pallas_call(
        functools.partial(_moe_body, n_exp=e, f_hid=f),
        grid=(n // _TN,),
        in_specs=[
            pl.BlockSpec((_TN, d), lambda i: (i, 0)),
            pl.BlockSpec((d, e), const),
            pl.BlockSpec((1, e), const),
            pl.BlockSpec((e, d, f), lambda i: (0, 0, 0)),
            pl.BlockSpec((1, ef), const),
            pl.BlockSpec((ef, d), const),
            pl.BlockSpec((e, d), const),
        ],
        out_specs=pl.BlockSpec((_TN, d), lambda i: (i, 0)),
        out_shape=jax.ShapeDtypeStruct((n, d), x.dtype),
        compiler_params=pltpu.CompilerParams(
            dimension_semantics=("parallel",)),
    )(x, Wg, bgr, W1, b1r, w2r, b2)
